# baseline (speedup 1.0000x reference)
import functools

import jax
import jax.numpy as jnp
from jax import lax
from jax.experimental import pallas as pl
from jax.experimental.pallas import tpu as pltpu
from jax.experimental.pallas import tpu_sc as plsc

VOCAB = 1000000
DIM = 64
NC = 2
NS = 16
NW = NC * NS

CHUNK = 128
B_TOK = 16384
L_SEQ = 50
NCHUNK_TOTAL = (B_TOK // CHUNK) * L_SEQ
NCH = NCHUNK_TOTAL // NW
BB_N = B_TOK // CHUNK


def _build_gather():
    mesh = plsc.VectorSubcoreMesh(core_axis_name="c", subcore_axis_name="s")

    @functools.partial(
        pl.kernel,
        mesh=mesh,
        out_type=jax.ShapeDtypeStruct((L_SEQ, DIM // 8, BB_N, 8, CHUNK),
                                      jnp.float32),
        compiler_params=pltpu.CompilerParams(use_tc_tiling_on_sc=False,
                                             needs_layout_passes=False),
        scratch_types=[
            pltpu.VMEM((NCH, CHUNK), jnp.int32),
            pltpu.VMEM((2, CHUNK, DIM), jnp.float32),
            pltpu.VMEM((2, DIM, CHUNK), jnp.float32),
            pltpu.SemaphoreType.DMA((2,)),
            pltpu.SemaphoreType.DMA((2,)),
        ],
    )
    def gather_kernel(idx_hbm, table_hbm, out_hbm, idx_v, bufs, tbufs,
                      gsem, wsem):
        c = lax.axis_index("c")
        s = lax.axis_index("s")
        wid = s * NC + c
        pltpu.sync_copy(idx_hbm.at[wid], idx_v)

        lane = lax.iota(jnp.int32, 16)
        rows = [lane + 16 * v for v in range(CHUNK // 16)]

        def fire_gather(j, slot):
            pltpu.async_copy(table_hbm.at[idx_v.at[j]], bufs.at[slot],
                             gsem.at[slot])

        def drain_gather(slot):
            pltpu.make_async_copy(table_hbm.at[pl.ds(0, CHUNK)],
                                  bufs.at[slot], gsem.at[slot]).wait()

        def fire_wb(j, slot):
            q = wid * NCH + j
            l = q // BB_N
            bb = lax.rem(q, BB_N)
            for tr in range(DIM // 8):
                pltpu.async_copy(tbufs.at[slot, pl.ds(8 * tr, 8)],
                                 out_hbm.at[l, tr, bb], wsem.at[slot])

        def drain_wb(slot):
            for tr in range(DIM // 8):
                pltpu.make_async_copy(tbufs.at[slot, pl.ds(0, 8)],
                                      out_hbm.at[0, 0, 0],
                                      wsem.at[slot]).wait()

        def transpose(slot):
            buf = bufs.at[slot]
            tbuf = tbufs.at[slot]

            @pl.loop(0, DIM, unroll=16)
            def _(d):
                col = jnp.full((16,), d, dtype=jnp.int32)
                for v in range(CHUNK // 16):
                    vals = plsc.load_gather(buf, [rows[v], col])
                    tbuf[d, pl.ds(16 * v, 16)] = vals

        fire_gather(0, 0)

        @pl.loop(0, NCH)
        def _(j):
            cur = lax.rem(j, 2)
            nxt = 1 - cur

            @pl.when(j + 1 < NCH)
            def _():
                fire_gather(j + 1, nxt)

            drain_gather(cur)

            @pl.when(j >= 2)
            def _():
                drain_wb(cur)

            transpose(cur)
            fire_wb(j, cur)

        drain_wb(0)
        drain_wb(1)

    return gather_kernel


_GATHER = _build_gather()


def kernel(x, embedding):
    idx = jnp.transpose(x).reshape(NW, NCH, CHUNK).astype(jnp.int32)
    out5 = _GATHER(idx, embedding)
    return out5.transpose(2, 4, 0, 1, 3).reshape(B_TOK, L_SEQ, DIM)


# speedup vs baseline: 1.4186x; 1.4186x over previous
import functools

import jax
import jax.numpy as jnp
from jax import lax
from jax.experimental import pallas as pl
from jax.experimental.pallas import tpu as pltpu
from jax.experimental.pallas import tpu_sc as plsc

VOCAB = 1000000
DIM = 64
NC = 2
NS = 16
NW = NC * NS

CHUNK = 128
B_TOK = 16384
L_SEQ = 50
NCHUNK_TOTAL = (B_TOK // CHUNK) * L_SEQ
NCH = NCHUNK_TOTAL // NW
BB_N = B_TOK // CHUNK


def _build_gather():
    mesh = plsc.VectorSubcoreMesh(core_axis_name="c", subcore_axis_name="s")

    @functools.partial(
        pl.kernel,
        mesh=mesh,
        out_type=jax.ShapeDtypeStruct((L_SEQ, DIM // 8, BB_N, 8, CHUNK),
                                      jnp.float32),
        compiler_params=pltpu.CompilerParams(use_tc_tiling_on_sc=False,
                                             needs_layout_passes=False),
        scratch_types=[
            pltpu.VMEM((NCH, CHUNK), jnp.int32),
            pltpu.VMEM((2, CHUNK, DIM), jnp.float32),
            pltpu.VMEM((2, DIM, CHUNK), jnp.float32),
            pltpu.SemaphoreType.DMA((2,)),
            pltpu.SemaphoreType.DMA((2,)),
        ],
    )
    def gather_kernel(idx_hbm, table_hbm, out_hbm, idx_v, bufs, tbufs,
                      gsem, wsem):
        c = lax.axis_index("c")
        s = lax.axis_index("s")
        wid = s * NC + c
        pltpu.sync_copy(idx_hbm.at[wid], idx_v)

        lane = lax.iota(jnp.int32, 16)
        rows = [lane + 16 * v for v in range(CHUNK // 16)]

        def fire_gather(j, slot):
            pltpu.async_copy(table_hbm.at[idx_v.at[j]], bufs.at[slot],
                             gsem.at[slot])

        def drain_gather(slot):
            pltpu.make_async_copy(table_hbm.at[pl.ds(0, CHUNK)],
                                  bufs.at[slot], gsem.at[slot]).wait()

        def fire_wb(j, slot):
            q = wid * NCH + j
            l = q // BB_N
            bb = lax.rem(q, BB_N)
            for tr in range(DIM // 8):
                pltpu.async_copy(tbufs.at[slot, pl.ds(8 * tr, 8)],
                                 out_hbm.at[l, tr, bb], wsem.at[slot])

        def drain_wb(slot):
            for tr in range(DIM // 8):
                pltpu.make_async_copy(tbufs.at[slot, pl.ds(0, 8)],
                                      out_hbm.at[0, 0, 0],
                                      wsem.at[slot]).wait()

        def transpose(slot):
            buf = bufs.at[slot]
            tbuf = tbufs.at[slot]

            @plsc.parallel_loop(0, DIM, unroll=8)
            def _(d):
                col = jnp.full((16,), d, dtype=jnp.int32)
                for v in range(CHUNK // 16):
                    vals = plsc.load_gather(buf, [rows[v], col])
                    tbuf[d, pl.ds(16 * v, 16)] = vals

        fire_gather(0, 0)

        @pl.loop(0, NCH // 2)
        def _(g):
            j0 = 2 * g
            j1 = 2 * g + 1

            fire_gather(j1, 1)
            drain_gather(0)

            @pl.when(g >= 1)
            def _():
                drain_wb(0)

            transpose(0)
            fire_wb(j0, 0)

            @pl.when(j1 + 1 < NCH)
            def _():
                fire_gather(j1 + 1, 0)

            drain_gather(1)

            @pl.when(g >= 1)
            def _():
                drain_wb(1)

            transpose(1)
            fire_wb(j1, 1)

        drain_wb(0)
        drain_wb(1)

    return gather_kernel


_GATHER = _build_gather()


def kernel(x, embedding):
    idx = jnp.transpose(x).reshape(NW, NCH, CHUNK).astype(jnp.int32)
    out5 = _GATHER(idx, embedding)
    return out5.transpose(2, 4, 0, 1, 3).reshape(B_TOK, L_SEQ, DIM)


# speedup vs baseline: 1.8546x; 1.3073x over previous
import functools

import jax
import jax.numpy as jnp
from jax import lax
from jax.experimental import pallas as pl
from jax.experimental.pallas import tpu as pltpu
from jax.experimental.pallas import tpu_sc as plsc

VOCAB = 1000000
DIM = 64
NC = 2
NS = 16
NW = NC * NS

CHUNK = 128
B_TOK = 16384
L_SEQ = 50
NCHUNK_TOTAL = (B_TOK // CHUNK) * L_SEQ
NCH = NCHUNK_TOTAL // NW
BB_N = B_TOK // CHUNK


def _build_gather():
    mesh = plsc.VectorSubcoreMesh(core_axis_name="c", subcore_axis_name="s")

    @functools.partial(
        pl.kernel,
        mesh=mesh,
        out_type=jax.ShapeDtypeStruct((L_SEQ, DIM // 8, BB_N, 8, CHUNK),
                                      jnp.float32),
        compiler_params=pltpu.CompilerParams(use_tc_tiling_on_sc=False,
                                             needs_layout_passes=False),
        scratch_types=[
            pltpu.VMEM((NCH, CHUNK), jnp.int32),
            pltpu.VMEM((2, CHUNK, DIM), jnp.float32),
            pltpu.VMEM((2, DIM, CHUNK), jnp.float32),
            pltpu.SemaphoreType.DMA((2,)),
            pltpu.SemaphoreType.DMA((2,)),
        ],
    )
    def gather_kernel(idx_hbm, table_hbm, out_hbm, idx_v, bufs, tbufs,
                      gsem, wsem):
        c = lax.axis_index("c")
        s = lax.axis_index("s")
        wid = s * NC + c
        pltpu.sync_copy(idx_hbm.at[wid], idx_v)

        lane = lax.iota(jnp.int32, 16)
        perms = [lax.rem(lane + k, 16) for k in range(16)]

        def fire_gather(j, slot):
            pltpu.async_copy(table_hbm.at[idx_v.at[j]], bufs.at[slot],
                             gsem.at[slot])

        def drain_gather(slot):
            pltpu.make_async_copy(table_hbm.at[pl.ds(0, CHUNK)],
                                  bufs.at[slot], gsem.at[slot]).wait()

        def fire_wb(j, slot):
            q = wid * NCH + j
            l = q // BB_N
            bb = lax.rem(q, BB_N)
            for tr in range(DIM // 8):
                pltpu.async_copy(tbufs.at[slot, pl.ds(8 * tr, 8)],
                                 out_hbm.at[l, tr, bb], wsem.at[slot])

        def drain_wb(slot):
            for tr in range(DIM // 8):
                pltpu.make_async_copy(tbufs.at[slot, pl.ds(0, 8)],
                                      out_hbm.at[0, 0, 0],
                                      wsem.at[slot]).wait()

        def transpose(slot):
            buf = bufs.at[slot]
            tbuf = tbufs.at[slot]

            @plsc.parallel_loop(0, CHUNK // 16, unroll=2)
            def _(v):
                row = lane + 16 * v
                for d0 in range(0, DIM, 16):
                    for k in range(16):
                        col = perms[k] + d0
                        vals = plsc.load_gather(buf, [row, col])
                        plsc.store_scatter(tbuf, [col, row], vals)

        fire_gather(0, 0)

        @pl.loop(0, NCH // 2)
        def _(g):
            j0 = 2 * g
            j1 = 2 * g + 1

            fire_gather(j1, 1)
            drain_gather(0)

            @pl.when(g >= 1)
            def _():
                drain_wb(0)

            transpose(0)
            fire_wb(j0, 0)

            @pl.when(j1 + 1 < NCH)
            def _():
                fire_gather(j1 + 1, 0)

            drain_gather(1)

            @pl.when(g >= 1)
            def _():
                drain_wb(1)

            transpose(1)
            fire_wb(j1, 1)

        drain_wb(0)
        drain_wb(1)

    return gather_kernel


_GATHER = _build_gather()


def kernel(x, embedding):
    idx = jnp.transpose(x).reshape(NW, NCH, CHUNK).astype(jnp.int32)
    out5 = _GATHER(idx, embedding)
    return out5.transpose(2, 4, 0, 1, 3).reshape(B_TOK, L_SEQ, DIM)


# speedup vs baseline: 1.9370x; 1.0445x over previous
import functools

import jax
import jax.numpy as jnp
from jax import lax
from jax.experimental import pallas as pl
from jax.experimental.pallas import tpu as pltpu
from jax.experimental.pallas import tpu_sc as plsc

VOCAB = 1000000
DIM = 64
NC = 2
NS = 16
NW = NC * NS

CHUNK = 128
B_TOK = 16384
L_SEQ = 50
NCHUNK_TOTAL = (B_TOK // CHUNK) * L_SEQ
NCH = NCHUNK_TOTAL // NW
BB_N = B_TOK // CHUNK


def _build_gather():
    mesh = plsc.VectorSubcoreMesh(core_axis_name="c", subcore_axis_name="s")

    @functools.partial(
        pl.kernel,
        mesh=mesh,
        out_type=jax.ShapeDtypeStruct((L_SEQ, DIM // 8, BB_N, 8, CHUNK),
                                      jnp.float32),
        compiler_params=pltpu.CompilerParams(use_tc_tiling_on_sc=False,
                                             needs_layout_passes=False),
        scratch_types=[
            pltpu.VMEM((NCH, CHUNK), jnp.int32),
            pltpu.VMEM((2, CHUNK, DIM), jnp.float32),
            pltpu.VMEM((2, DIM, CHUNK), jnp.float32),
            pltpu.SemaphoreType.DMA((2,)),
            pltpu.SemaphoreType.DMA((2,)),
        ],
    )
    def gather_kernel(idx_hbm, table_hbm, out_hbm, idx_v, bufs, tbufs,
                      gsem, wsem):
        c = lax.axis_index("c")
        s = lax.axis_index("s")
        wid = s * NC + c
        pltpu.sync_copy(idx_hbm.at[wid], idx_v)

        lane = lax.iota(jnp.int32, 16)
        perms = [lax.rem(lane + k, 16) for k in range(16)]

        def fire_gather(j, slot):
            pltpu.async_copy(table_hbm.at[idx_v.at[j]], bufs.at[slot],
                             gsem.at[slot])

        def drain_gather(slot):
            pltpu.make_async_copy(table_hbm.at[pl.ds(0, CHUNK)],
                                  bufs.at[slot], gsem.at[slot]).wait()

        def fire_wb(j, slot):
            q = wid * NCH + j
            l = q // BB_N
            bb = lax.rem(q, BB_N)
            for tr in range(DIM // 8):
                pltpu.async_copy(tbufs.at[slot, pl.ds(8 * tr, 8)],
                                 out_hbm.at[l, tr, bb], wsem.at[slot])

        def drain_wb(slot):
            for tr in range(DIM // 8):
                pltpu.make_async_copy(tbufs.at[slot, pl.ds(0, 8)],
                                      out_hbm.at[0, 0, 0],
                                      wsem.at[slot]).wait()

        def transpose(slot):
            buf = bufs.at[slot]
            tbuf = tbufs.at[slot]

            @plsc.parallel_loop(0, DIM // 16, unroll=2)
            def _(b):
                d0 = 16 * b
                cols = [perms[k] + d0 for k in range(16)]
                for v in range(CHUNK // 16):
                    row = lane + 16 * v
                    for k in range(16):
                        vals = plsc.load_gather(buf, [row, cols[k]])
                        plsc.store_scatter(tbuf, [cols[k], row], vals)

        fire_gather(0, 0)

        @pl.loop(0, NCH // 2)
        def _(g):
            j0 = 2 * g
            j1 = 2 * g + 1

            fire_gather(j1, 1)
            drain_gather(0)

            @pl.when(g >= 1)
            def _():
                drain_wb(0)

            transpose(0)
            fire_wb(j0, 0)

            @pl.when(j1 + 1 < NCH)
            def _():
                fire_gather(j1 + 1, 0)

            drain_gather(1)

            @pl.when(g >= 1)
            def _():
                drain_wb(1)

            transpose(1)
            fire_wb(j1, 1)

        drain_wb(0)
        drain_wb(1)

    return gather_kernel


_GATHER = _build_gather()


def kernel(x, embedding):
    idx = jnp.transpose(x).reshape(NW, NCH, CHUNK).astype(jnp.int32)
    out5 = _GATHER(idx, embedding)
    return out5.transpose(2, 4, 0, 1, 3).reshape(B_TOK, L_SEQ, DIM)


# speedup vs baseline: 2.6758x; 1.3814x over previous
import functools

import jax
import jax.numpy as jnp
from jax import lax
from jax.experimental import pallas as pl
from jax.experimental.pallas import tpu as pltpu
from jax.experimental.pallas import tpu_sc as plsc

VOCAB = 1000000
DIM = 64
NC = 2
NS = 16
NW = NC * NS

CHUNK = 128
B_TOK = 16384
L_SEQ = 50
NCH = (B_TOK // CHUNK) * L_SEQ // NW
BB_N = B_TOK // CHUNK

VBLK = VOCAB // CHUNK
T2_ROWS = 500032
BLK_W = CHUNK * DIM
NJ1 = 2 * ((VBLK + 2 * NW - 1) // (2 * NW))


def _build_retile():
    mesh = plsc.VectorSubcoreMesh(core_axis_name="c", subcore_axis_name="s")

    @functools.partial(
        pl.kernel,
        mesh=mesh,
        out_type=jax.ShapeDtypeStruct((T2_ROWS, 2 * DIM), jnp.float32),
        compiler_params=pltpu.CompilerParams(use_tc_tiling_on_sc=True,
                                             needs_layout_passes=False),
        scratch_types=[
            pltpu.VMEM((2, DIM, CHUNK), jnp.float32),
            pltpu.VMEM((2, DIM, 2 * DIM), jnp.float32),
            pltpu.VMEM((32, 2 * DIM), jnp.float32),
            pltpu.SemaphoreType.DMA((2,)),
            pltpu.SemaphoreType.DMA((2,)),
        ],
    )
    def retile_kernel(tt_hbm, tail_hbm, t2_hbm, inbufs, tbufs, tstage,
                      gsem, wsem):
        c = lax.axis_index("c")
        s = lax.axis_index("s")
        wid = s * NC + c

        lane = lax.iota(jnp.int32, 16)
        perms = [(lane + k) & 15 for k in range(16)]

        def blk(j):
            return wid + NW * j

        def fire_in(j, slot):
            k = blk(j)
            for tj in range(DIM // 8):
                pltpu.async_copy(
                    tt_hbm.at[pl.ds(8 * tj, 8), pl.ds(k * CHUNK, CHUNK)],
                    inbufs.at[slot, pl.ds(8 * tj, 8)], gsem.at[slot])

        def drain_in(slot):
            pltpu.make_async_copy(
                tt_hbm.at[pl.ds(0, DIM), pl.ds(0, CHUNK)],
                inbufs.at[slot], gsem.at[slot]).wait()

        def fire_out(j, slot):
            pltpu.async_copy(tbufs.at[slot],
                             t2_hbm.at[pl.ds(blk(j) * DIM, DIM)],
                             wsem.at[slot])

        def drain_out(slot):
            pltpu.make_async_copy(tbufs.at[slot],
                                  t2_hbm.at[pl.ds(0, DIM)],
                                  wsem.at[slot]).wait()

        def transpose(slot):
            inbuf = inbufs.at[slot]
            tbuf = tbufs.at[slot]

            @plsc.parallel_loop(0, DIM // 16, unroll=2)
            def _(b):
                d0 = 16 * b
                dcols = [perms[k] + d0 for k in range(16)]
                for cb in range(CHUNK // 16):
                    rowc = lane + 16 * cb
                    rowc64 = rowc * DIM
                    for k in range(16):
                        w = rowc64 + dcols[k]
                        vals = plsc.load_gather(inbuf, [dcols[k], rowc])
                        plsc.store_scatter(
                            tbuf, [lax.shift_right_logical(w, 7), w & 127],
                            vals)

        def step(j, slot):
            @pl.when(blk(j) < VBLK)
            def _():
                drain_in(slot)

                @pl.when(j >= 2)
                def _():
                    drain_out(slot)

                transpose(slot)
                fire_out(j, slot)

            @pl.when(blk(j + 2) < VBLK)
            def _():
                fire_in(j + 2, slot)

        @pl.when(blk(0) < VBLK)
        def _():
            fire_in(0, 0)

        @pl.when(blk(1) < VBLK)
        def _():
            fire_in(1, 1)

        @pl.loop(0, NJ1 // 2)
        def _(g):
            step(2 * g, 0)
            step(2 * g + 1, 1)

        for j in range(NJ1 - 4, NJ1):
            fired = blk(j) < VBLK
            undrained = True if j + 2 >= NJ1 else blk(j + 2) >= VBLK

            @pl.when(jnp.logical_and(fired, undrained))
            def _():
                drain_out(j % 2)

        @pl.when(wid == 4)
        def _():
            pltpu.sync_copy(tail_hbm, tstage)
            pltpu.sync_copy(tstage, t2_hbm.at[pl.ds(VBLK * DIM, 32)])

    return retile_kernel


def _build_gather():
    mesh = plsc.VectorSubcoreMesh(core_axis_name="c", subcore_axis_name="s")

    @functools.partial(
        pl.kernel,
        mesh=mesh,
        out_type=jax.ShapeDtypeStruct((L_SEQ, DIM // 8, BB_N, 8, CHUNK),
                                      jnp.float32),
        compiler_params=pltpu.CompilerParams(use_tc_tiling_on_sc=True,
                                             needs_layout_passes=False),
        scratch_types=[
            pltpu.VMEM((NCH, CHUNK), jnp.int32),
            pltpu.VMEM((NCH, CHUNK), jnp.int32),
            pltpu.VMEM((2, CHUNK, 2 * DIM), jnp.float32),
            pltpu.VMEM((2, DIM, CHUNK), jnp.float32),
            pltpu.SemaphoreType.DMA((2,)),
            pltpu.SemaphoreType.DMA((2,)),
        ],
    )
    def gather_kernel(idx_hbm, t2_hbm, out_hbm, idx_v, hoff_v, bufs, tbufs,
                      gsem, wsem):
        c = lax.axis_index("c")
        s = lax.axis_index("s")
        wid = s * NC + c
        pltpu.sync_copy(idx_hbm.at[wid], idx_v)

        lane = lax.iota(jnp.int32, 16)
        perms = [(lane + k) & 15 for k in range(16)]

        @plsc.parallel_loop(0, NCH, unroll=4)
        def _(j):
            for v in range(CHUNK // 16):
                x = idx_v[j, pl.ds(16 * v, 16)]
                hoff_v[j, pl.ds(16 * v, 16)] = (x & 1) * DIM
                idx_v[j, pl.ds(16 * v, 16)] = lax.shift_right_logical(x, 1)

        def fire_gather(j, slot):
            pltpu.async_copy(t2_hbm.at[idx_v.at[j]], bufs.at[slot],
                             gsem.at[slot])

        def drain_gather(slot):
            pltpu.make_async_copy(t2_hbm.at[pl.ds(0, CHUNK)],
                                  bufs.at[slot], gsem.at[slot]).wait()

        def fire_wb(j, slot):
            q = wid * NCH + j
            l = q // BB_N
            bb = lax.rem(q, BB_N)
            for tr in range(DIM // 8):
                pltpu.async_copy(tbufs.at[slot, pl.ds(8 * tr, 8)],
                                 out_hbm.at[l, tr, bb], wsem.at[slot])

        def drain_wb(slot):
            for tr in range(DIM // 8):
                pltpu.make_async_copy(tbufs.at[slot, pl.ds(0, 8)],
                                      out_hbm.at[0, 0, 0],
                                      wsem.at[slot]).wait()

        def transpose(j, slot):
            buf = bufs.at[slot]
            tbuf = tbufs.at[slot]

            @plsc.parallel_loop(0, DIM // 16, unroll=2)
            def _(b):
                d0 = 16 * b
                dcols = [perms[k] + d0 for k in range(16)]
                for v in range(CHUNK // 16):
                    row = lane + 16 * v
                    hv = hoff_v[j, pl.ds(16 * v, 16)]
                    for k in range(16):
                        vals = plsc.load_gather(buf, [row, hv + dcols[k]])
                        plsc.store_scatter(tbuf, [dcols[k], row], vals)

        fire_gather(0, 0)

        @pl.loop(0, NCH // 2)
        def _(g):
            j0 = 2 * g
            j1 = 2 * g + 1

            fire_gather(j1, 1)
            drain_gather(0)

            @pl.when(g >= 1)
            def _():
                drain_wb(0)

            transpose(j0, 0)
            fire_wb(j0, 0)

            @pl.when(j1 + 1 < NCH)
            def _():
                fire_gather(j1 + 1, 0)

            drain_gather(1)

            @pl.when(g >= 1)
            def _():
                drain_wb(1)

            transpose(j1, 1)
            fire_wb(j1, 1)

        drain_wb(0)
        drain_wb(1)

    return gather_kernel


_RETILE = _build_retile()
_GATHER = _build_gather()


def kernel(x, embedding):
    idx = jnp.transpose(x).reshape(NW, NCH, CHUNK).astype(jnp.int32)
    tail = embedding[VBLK * CHUNK:].reshape(32, 2 * DIM)
    table2 = _RETILE(jnp.transpose(embedding), tail)
    out5 = _GATHER(idx, table2)
    return out5.transpose(2, 4, 0, 1, 3).reshape(B_TOK, L_SEQ, DIM)


# speedup vs baseline: 2.7848x; 1.0407x over previous
import functools

import jax
import jax.numpy as jnp
from jax import lax
from jax.experimental import pallas as pl
from jax.experimental.pallas import tpu as pltpu
from jax.experimental.pallas import tpu_sc as plsc

VOCAB = 1000000
DIM = 64
NC = 2
NS = 16
NW = NC * NS

CHUNK = 128
B_TOK = 16384
L_SEQ = 50
NCH = (B_TOK // CHUNK) * L_SEQ // NW
BB_N = B_TOK // CHUNK

VBLK = VOCAB // CHUNK
T2_ROWS = 500032
BLK_W = CHUNK * DIM
NJ1 = 2 * ((VBLK + 2 * NW - 1) // (2 * NW))


def _build_retile():
    mesh = plsc.VectorSubcoreMesh(core_axis_name="c", subcore_axis_name="s")

    @functools.partial(
        pl.kernel,
        mesh=mesh,
        out_type=jax.ShapeDtypeStruct((T2_ROWS, 2 * DIM), jnp.float32),
        compiler_params=pltpu.CompilerParams(use_tc_tiling_on_sc=True,
                                             needs_layout_passes=False),
        scratch_types=[
            pltpu.VMEM((2, DIM, CHUNK), jnp.float32),
            pltpu.VMEM((2, DIM, 2 * DIM), jnp.float32),
            pltpu.VMEM((32, 2 * DIM), jnp.float32),
            pltpu.SemaphoreType.DMA((2,)),
            pltpu.SemaphoreType.DMA((2,)),
        ],
    )
    def retile_kernel(tt_hbm, tail_hbm, t2_hbm, inbufs, tbufs, tstage,
                      gsem, wsem):
        c = lax.axis_index("c")
        s = lax.axis_index("s")
        wid = s * NC + c

        lane = lax.iota(jnp.int32, 16)
        perms = [(lane + k) & 15 for k in range(16)]

        def blk(j):
            return wid + NW * j

        def fire_in(j, slot):
            k = blk(j)
            for tj in range(DIM // 8):
                pltpu.async_copy(
                    tt_hbm.at[pl.ds(8 * tj, 8), pl.ds(k * CHUNK, CHUNK)],
                    inbufs.at[slot, pl.ds(8 * tj, 8)], gsem.at[slot])

        def drain_in(slot):
            pltpu.make_async_copy(
                tt_hbm.at[pl.ds(0, DIM), pl.ds(0, CHUNK)],
                inbufs.at[slot], gsem.at[slot]).wait()

        def fire_out(j, slot):
            pltpu.async_copy(tbufs.at[slot],
                             t2_hbm.at[pl.ds(blk(j) * DIM, DIM)],
                             wsem.at[slot])

        def drain_out(slot):
            pltpu.make_async_copy(tbufs.at[slot],
                                  t2_hbm.at[pl.ds(0, DIM)],
                                  wsem.at[slot]).wait()

        def transpose(slot):
            inbuf = inbufs.at[slot]
            tbuf = tbufs.at[slot]

            @plsc.parallel_loop(0, DIM // 16, unroll=2)
            def _(b):
                d0 = 16 * b
                dcols = [perms[k] + d0 for k in range(16)]
                for cb in range(CHUNK // 16):
                    rowc = lane + 16 * cb
                    rowc64 = rowc * DIM
                    for k in range(16):
                        w = rowc64 + dcols[k]
                        vals = plsc.load_gather(inbuf, [dcols[k], rowc])
                        plsc.store_scatter(
                            tbuf, [lax.shift_right_logical(w, 7), w & 127],
                            vals)

        def step(j, slot):
            @pl.when(blk(j) < VBLK)
            def _():
                drain_in(slot)

                @pl.when(j >= 2)
                def _():
                    drain_out(slot)

                transpose(slot)
                fire_out(j, slot)

            @pl.when(blk(j + 2) < VBLK)
            def _():
                fire_in(j + 2, slot)

        @pl.when(blk(0) < VBLK)
        def _():
            fire_in(0, 0)

        @pl.when(blk(1) < VBLK)
        def _():
            fire_in(1, 1)

        @pl.loop(0, NJ1 // 2)
        def _(g):
            step(2 * g, 0)
            step(2 * g + 1, 1)

        for j in range(NJ1 - 4, NJ1):
            fired = blk(j) < VBLK
            undrained = True if j + 2 >= NJ1 else blk(j + 2) >= VBLK

            @pl.when(jnp.logical_and(fired, undrained))
            def _():
                drain_out(j % 2)

        @pl.when(wid == 4)
        def _():
            pltpu.sync_copy(tail_hbm, tstage)
            pltpu.sync_copy(tstage, t2_hbm.at[pl.ds(VBLK * DIM, 32)])

    return retile_kernel


def _build_gather():
    mesh = plsc.VectorSubcoreMesh(core_axis_name="c", subcore_axis_name="s")

    @functools.partial(
        pl.kernel,
        mesh=mesh,
        out_type=jax.ShapeDtypeStruct((L_SEQ, DIM // 8, BB_N, 8, CHUNK),
                                      jnp.float32),
        compiler_params=pltpu.CompilerParams(use_tc_tiling_on_sc=False,
                                             needs_layout_passes=False),
        scratch_types=[
            pltpu.VMEM((NCH, CHUNK), jnp.int32),
            pltpu.VMEM((2, CHUNK, DIM), jnp.float32),
            pltpu.VMEM((2, DIM, CHUNK), jnp.float32),
            pltpu.SemaphoreType.DMA((2,)),
            pltpu.SemaphoreType.DMA((2,)),
        ],
    )
    def gather_kernel(idx_hbm, t2_hbm, out_hbm, idx_v, bufs, tbufs,
                      gsem, wsem):
        c = lax.axis_index("c")
        s = lax.axis_index("s")
        wid = s * NC + c
        pltpu.sync_copy(idx_hbm.at[wid], idx_v)

        lane = lax.iota(jnp.int32, 16)
        perms = [(lane + k) & 15 for k in range(16)]

        def fire_gather(j, slot):
            pltpu.async_copy(t2_hbm.at[idx_v.at[j]], bufs.at[slot],
                             gsem.at[slot])

        def drain_gather(slot):
            pltpu.make_async_copy(t2_hbm.at[pl.ds(0, CHUNK)],
                                  bufs.at[slot], gsem.at[slot]).wait()

        def fire_wb(j, slot):
            q = wid * NCH + j
            l = q // BB_N
            bb = lax.rem(q, BB_N)
            for tr in range(DIM // 8):
                pltpu.async_copy(tbufs.at[slot, pl.ds(8 * tr, 8)],
                                 out_hbm.at[l, tr, bb], wsem.at[slot])

        def drain_wb(slot):
            for tr in range(DIM // 8):
                pltpu.make_async_copy(tbufs.at[slot, pl.ds(0, 8)],
                                      out_hbm.at[0, 0, 0],
                                      wsem.at[slot]).wait()

        def transpose(j, slot):
            buf = bufs.at[slot]
            tbuf = tbufs.at[slot]

            @plsc.parallel_loop(0, DIM // 16, unroll=2)
            def _(b):
                d0 = 16 * b
                dcols = [perms[k] + d0 for k in range(16)]
                for v in range(CHUNK // 16):
                    row = lane + 16 * v
                    for k in range(16):
                        vals = plsc.load_gather(buf, [row, dcols[k]])
                        plsc.store_scatter(tbuf, [dcols[k], row], vals)

        fire_gather(0, 0)

        @pl.loop(0, NCH // 2)
        def _(g):
            j0 = 2 * g
            j1 = 2 * g + 1

            fire_gather(j1, 1)
            drain_gather(0)

            @pl.when(g >= 1)
            def _():
                drain_wb(0)

            transpose(j0, 0)
            fire_wb(j0, 0)

            @pl.when(j1 + 1 < NCH)
            def _():
                fire_gather(j1 + 1, 0)

            drain_gather(1)

            @pl.when(g >= 1)
            def _():
                drain_wb(1)

            transpose(j1, 1)
            fire_wb(j1, 1)

        drain_wb(0)
        drain_wb(1)

    return gather_kernel


_RETILE = _build_retile()
_GATHER = _build_gather()


def kernel(x, embedding):
    idx = jnp.transpose(x).reshape(NW, NCH, CHUNK).astype(jnp.int32)
    tail = embedding[VBLK * CHUNK:].reshape(32, 2 * DIM)
    table2 = _RETILE(jnp.transpose(embedding), tail)
    out5 = _GATHER(idx, table2.reshape(2 * T2_ROWS, DIM))
    return out5.transpose(2, 4, 0, 1, 3).reshape(B_TOK, L_SEQ, DIM)


# speedup vs baseline: 3.1374x; 1.1266x over previous
import functools

import jax
import jax.numpy as jnp
from jax import lax
from jax.experimental import pallas as pl
from jax.experimental.pallas import tpu as pltpu
from jax.experimental.pallas import tpu_sc as plsc

VOCAB = 1000000
DIM = 64
NC = 2
NS = 16
NW = NC * NS

CHUNK = 128
B_TOK = 16384
L_SEQ = 50
NCH = (B_TOK // CHUNK) * L_SEQ // NW
BB_N = B_TOK // CHUNK

VBLK = VOCAB // CHUNK
T2_ROWS = 500032
BLK_W = CHUNK * DIM
NJ1 = 2 * ((VBLK + 2 * NW - 1) // (2 * NW))


def _build_retile():
    mesh = plsc.VectorSubcoreMesh(core_axis_name="c", subcore_axis_name="s")

    @functools.partial(
        pl.kernel,
        mesh=mesh,
        out_type=jax.ShapeDtypeStruct((T2_ROWS, 2 * DIM), jnp.float32),
        compiler_params=pltpu.CompilerParams(use_tc_tiling_on_sc=True,
                                             needs_layout_passes=False),
        scratch_types=[
            pltpu.VMEM((2, DIM, CHUNK), jnp.float32),
            pltpu.VMEM((2, DIM, 2 * DIM), jnp.float32),
            pltpu.VMEM((32, 2 * DIM), jnp.float32),
            pltpu.SemaphoreType.DMA((2,)),
            pltpu.SemaphoreType.DMA((2,)),
        ],
    )
    def retile_kernel(tt_hbm, tail_hbm, t2_hbm, inbufs, tbufs, tstage,
                      gsem, wsem):
        c = lax.axis_index("c")
        s = lax.axis_index("s")
        wid = s * NC + c

        lane = lax.iota(jnp.int32, 16)
        perms = [(lane + k) & 15 for k in range(16)]

        def blk(j):
            return wid + NW * j

        def fire_in(j, slot):
            k = blk(j)
            for tj in range(DIM // 8):
                pltpu.async_copy(
                    tt_hbm.at[pl.ds(8 * tj, 8), pl.ds(k * CHUNK, CHUNK)],
                    inbufs.at[slot, pl.ds(8 * tj, 8)], gsem.at[slot])

        def drain_in(slot):
            pltpu.make_async_copy(
                tt_hbm.at[pl.ds(0, DIM), pl.ds(0, CHUNK)],
                inbufs.at[slot], gsem.at[slot]).wait()

        def fire_out(j, slot):
            pltpu.async_copy(tbufs.at[slot],
                             t2_hbm.at[pl.ds(blk(j) * DIM, DIM)],
                             wsem.at[slot])

        def drain_out(slot):
            pltpu.make_async_copy(tbufs.at[slot],
                                  t2_hbm.at[pl.ds(0, DIM)],
                                  wsem.at[slot]).wait()

        def transpose(slot):
            inbuf = inbufs.at[slot]
            tbuf = tbufs.at[slot]

            @plsc.parallel_loop(0, DIM // 16, unroll=2)
            def _(b):
                d0 = 16 * b
                dcols = [perms[k] + d0 for k in range(16)]
                for cb in range(CHUNK // 16):
                    rowc = lane + 16 * cb
                    rowc64 = rowc * DIM
                    vals = [plsc.load_gather(inbuf, [dcols[k], rowc])
                            for k in range(16)]
                    for k in range(16):
                        w = rowc64 + dcols[k]
                        plsc.store_scatter(
                            tbuf, [lax.shift_right_logical(w, 7), w & 127],
                            vals[k])

        def step(j, slot):
            @pl.when(blk(j) < VBLK)
            def _():
                drain_in(slot)

                @pl.when(j >= 2)
                def _():
                    drain_out(slot)

                transpose(slot)
                fire_out(j, slot)

            @pl.when(blk(j + 2) < VBLK)
            def _():
                fire_in(j + 2, slot)

        @pl.when(blk(0) < VBLK)
        def _():
            fire_in(0, 0)

        @pl.when(blk(1) < VBLK)
        def _():
            fire_in(1, 1)

        @pl.loop(0, NJ1 // 2)
        def _(g):
            step(2 * g, 0)
            step(2 * g + 1, 1)

        for j in range(NJ1 - 4, NJ1):
            fired = blk(j) < VBLK
            undrained = True if j + 2 >= NJ1 else blk(j + 2) >= VBLK

            @pl.when(jnp.logical_and(fired, undrained))
            def _():
                drain_out(j % 2)

        @pl.when(wid == 4)
        def _():
            pltpu.sync_copy(tail_hbm, tstage)
            pltpu.sync_copy(tstage, t2_hbm.at[pl.ds(VBLK * DIM, 32)])

    return retile_kernel


def _build_gather():
    mesh = plsc.VectorSubcoreMesh(core_axis_name="c", subcore_axis_name="s")

    @functools.partial(
        pl.kernel,
        mesh=mesh,
        out_type=jax.ShapeDtypeStruct((L_SEQ, DIM // 8, BB_N, 8, CHUNK),
                                      jnp.float32),
        compiler_params=pltpu.CompilerParams(use_tc_tiling_on_sc=False,
                                             needs_layout_passes=False),
        scratch_types=[
            pltpu.VMEM((NCH, CHUNK), jnp.int32),
            pltpu.VMEM((2, CHUNK, DIM), jnp.float32),
            pltpu.VMEM((2, DIM, CHUNK), jnp.float32),
            pltpu.SemaphoreType.DMA((2,)),
            pltpu.SemaphoreType.DMA((2,)),
        ],
    )
    def gather_kernel(idx_hbm, t2_hbm, out_hbm, idx_v, bufs, tbufs,
                      gsem, wsem):
        c = lax.axis_index("c")
        s = lax.axis_index("s")
        wid = s * NC + c
        pltpu.sync_copy(idx_hbm.at[wid], idx_v)

        lane = lax.iota(jnp.int32, 16)
        perms = [(lane + k) & 15 for k in range(16)]

        def fire_gather(j, slot):
            pltpu.async_copy(t2_hbm.at[idx_v.at[j]], bufs.at[slot],
                             gsem.at[slot])

        def drain_gather(slot):
            pltpu.make_async_copy(t2_hbm.at[pl.ds(0, CHUNK)],
                                  bufs.at[slot], gsem.at[slot]).wait()

        def fire_wb(j, slot):
            q = wid * NCH + j
            l = q // BB_N
            bb = lax.rem(q, BB_N)
            for tr in range(DIM // 8):
                pltpu.async_copy(tbufs.at[slot, pl.ds(8 * tr, 8)],
                                 out_hbm.at[l, tr, bb], wsem.at[slot])

        def drain_wb(slot):
            for tr in range(DIM // 8):
                pltpu.make_async_copy(tbufs.at[slot, pl.ds(0, 8)],
                                      out_hbm.at[0, 0, 0],
                                      wsem.at[slot]).wait()

        def transpose(j, slot):
            buf = bufs.at[slot]
            tbuf = tbufs.at[slot]

            @plsc.parallel_loop(0, DIM // 16, unroll=2)
            def _(b):
                d0 = 16 * b
                dcols = [perms[k] + d0 for k in range(16)]
                for v in range(CHUNK // 16):
                    row = lane + 16 * v
                    vals = [plsc.load_gather(buf, [row, dcols[k]])
                            for k in range(16)]
                    for k in range(16):
                        plsc.store_scatter(tbuf, [dcols[k], row], vals[k])

        fire_gather(0, 0)

        @pl.loop(0, NCH // 2)
        def _(g):
            j0 = 2 * g
            j1 = 2 * g + 1

            fire_gather(j1, 1)
            drain_gather(0)

            @pl.when(g >= 1)
            def _():
                drain_wb(0)

            transpose(j0, 0)
            fire_wb(j0, 0)

            @pl.when(j1 + 1 < NCH)
            def _():
                fire_gather(j1 + 1, 0)

            drain_gather(1)

            @pl.when(g >= 1)
            def _():
                drain_wb(1)

            transpose(j1, 1)
            fire_wb(j1, 1)

        drain_wb(0)
        drain_wb(1)

    return gather_kernel


_RETILE = _build_retile()
_GATHER = _build_gather()


def kernel(x, embedding):
    idx = jnp.transpose(x).reshape(NW, NCH, CHUNK).astype(jnp.int32)
    tail = embedding[VBLK * CHUNK:].reshape(32, 2 * DIM)
    table2 = _RETILE(jnp.transpose(embedding), tail)
    out5 = _GATHER(idx, table2.reshape(2 * T2_ROWS, DIM))
    return out5.transpose(2, 4, 0, 1, 3).reshape(B_TOK, L_SEQ, DIM)
